# baseline (speedup 1.0000x reference)
import jax
import jax.numpy as jnp
from jax import lax
from jax.experimental import pallas as pl
from jax.experimental.pallas import tpu as pltpu
from jax.experimental.pallas import tpu_sc as plsc

N = 10000
E = 320000
D = 128
NC, NS = 2, 16

CH = 80
EROWS = 4096
EPAD = EROWS * CH - E
MROWS = EROWS // NS
DROWS = EROWS // (NC * NS)
JUNK = 10239

ACC_N = 10240
RPT = ACC_N // NS
HB = ACC_N // D

_mesh = plsc.VectorSubcoreMesh(core_axis_name="c", subcore_axis_name="s")


def _deg_body(dst2, ones_hbm, zero128_hbm, out, acc, idx_d, ones_v):
    c = lax.axis_index("c")
    s = lax.axis_index("s")
    w = c * NS + s

    def run(out_view):
        pltpu.sync_copy(zero128_hbm, ones_v)
        for j in range(RPT // CH):
            pltpu.sync_copy(ones_v, acc.at[pl.ds(s * RPT + j * CH, CH)])
        pltpu.sync_copy(ones_hbm, ones_v)
        pltpu.sync_copy(dst2.at[pl.ds(w * DROWS, DROWS)], idx_d)
        plsc.subcore_barrier()

        def chunk(k, carry):
            pltpu.sync_copy(ones_v, acc.at[idx_d.at[k]], add=True)
            return carry

        lax.fori_loop(0, DROWS, chunk, 0)
        plsc.subcore_barrier()
        zbuf = ones_v

        @pl.when(s < NS - 1)
        def _():
            for j in range(RPT // CH):
                r = s * RPT + j * CH
                pltpu.sync_copy(acc.at[pl.ds(r, CH)], zbuf)
                pltpu.sync_copy(zbuf, out_view.at[pl.ds(r, CH)])

        @pl.when(s == NS - 1)
        def _():
            for j in range((N - (NS - 1) * RPT) // CH):
                r = (NS - 1) * RPT + j * CH
                pltpu.sync_copy(acc.at[pl.ds(r, CH)], zbuf)
                pltpu.sync_copy(zbuf, out_view.at[pl.ds(r, CH)])

    @pl.when(c == 0)
    def _():
        run(out.at[0])

    @pl.when(c == 1)
    def _():
        run(out.at[1])


def _sc_degree(dst2, ones128, zero128):
    return pl.kernel(
        _deg_body,
        out_type=jax.ShapeDtypeStruct((NC, N, D), jnp.float32),
        mesh=_mesh,
        scratch_types=[
            pltpu.VMEM_SHARED((ACC_N, D), jnp.float32),
            pltpu.VMEM((DROWS, CH), jnp.int32),
            pltpu.VMEM((CH, D), jnp.float32),
        ],
    )(dst2, ones128, zero128)


def _perm_body(h1, perm2, out, buf, idx_v):
    c = lax.axis_index("c")
    s = lax.axis_index("s")

    def gather_chunks(nchunks):
        pltpu.sync_copy(perm2.at[pl.ds(s * 8, 8)], idx_v)
        for j in range(nchunks):
            pltpu.sync_copy(h1.at[idx_v.at[j]], buf)
            pltpu.sync_copy(buf, out.at[pl.ds(s * 8 * CH + j * CH, CH)])

    @pl.when(c == 0)
    def _():
        @pl.when(s < NS - 1)
        def _():
            gather_chunks(8)

        @pl.when(s == NS - 1)
        def _():
            gather_chunks(5)


def _sc_perm(h1, perm2):
    return pl.kernel(
        _perm_body,
        out_type=jax.ShapeDtypeStruct((N, D), jnp.float32),
        mesh=_mesh,
        scratch_types=[
            pltpu.VMEM((CH, D), jnp.float32),
            pltpu.VMEM((8, CH), jnp.int32),
        ],
    )(h1, perm2)


def _msg_body(tbl, src2, dst2, zero128_hbm, out, acc, idx_s, idx_d, buf, gsem):
    c = lax.axis_index("c")
    s = lax.axis_index("s")

    def seg(tbl_view, row0, nrows):
        pltpu.sync_copy(src2.at[pl.ds(row0, nrows)], idx_s.at[pl.ds(0, nrows)])
        pltpu.sync_copy(dst2.at[pl.ds(row0, nrows)], idx_d.at[pl.ds(0, nrows)])
        pltpu.async_copy(tbl_view.at[idx_s.at[0]], buf.at[0], gsem)

        def chunk(k, carry):
            @pl.when(k + 1 < nrows)
            def _():
                pltpu.async_copy(tbl_view.at[idx_s.at[k + 1]],
                                 buf.at[(k + 1) % 2], gsem)
            pltpu.make_async_copy(tbl_view.at[idx_s.at[k]],
                                  buf.at[k % 2], gsem).wait()
            pltpu.sync_copy(buf.at[k % 2], acc.at[idx_d.at[k]], add=True)
            return carry

        lax.fori_loop(0, nrows, chunk, 0)

    def run(tbl_view, out_view):
        zbuf = buf.at[0]
        pltpu.sync_copy(zero128_hbm, zbuf)
        for j in range(RPT // CH):
            pltpu.sync_copy(zbuf, acc.at[pl.ds(s * RPT + j * CH, CH)])
        plsc.subcore_barrier()

        for j in range(MROWS // 64):
            seg(tbl_view, s * MROWS + 64 * j, 64)

        plsc.subcore_barrier()
        @pl.when(s < NS - 1)
        def _():
            for j in range(RPT // CH):
                r = s * RPT + j * CH
                pltpu.sync_copy(acc.at[pl.ds(r, CH)], zbuf)
                pltpu.sync_copy(zbuf, out_view.at[pl.ds(r, CH)])

        @pl.when(s == NS - 1)
        def _():
            for j in range((N - (NS - 1) * RPT) // CH):
                r = (NS - 1) * RPT + j * CH
                pltpu.sync_copy(acc.at[pl.ds(r, CH)], zbuf)
                pltpu.sync_copy(zbuf, out_view.at[pl.ds(r, CH)])

    @pl.when(c == 0)
    def _():
        run(tbl.at[0], out.at[0])

    @pl.when(c == 1)
    def _():
        run(tbl.at[1], out.at[1])


def _sc_msg(tbl, src2, dst2, zero128):
    return pl.kernel(
        _msg_body,
        out_type=jax.ShapeDtypeStruct((NC, N, D), jnp.float32),
        mesh=_mesh,
        scratch_types=[
            pltpu.VMEM_SHARED((ACC_N, D), jnp.float32),
            pltpu.VMEM((64, CH), jnp.int32),
            pltpu.VMEM((64, CH), jnp.int32),
            pltpu.VMEM((2, CH, D), jnp.float32),
            pltpu.SemaphoreType.DMA,
        ],
    )(tbl, src2, dst2, zero128)


def _prep_body(cnt_ref, dis_ref):
    deg = cnt_ref[0, :, :1] + cnt_ref[1, :, :1] + 1.0
    dis_ref[...] = lax.rsqrt(deg)


def _tc_prep(counts):
    return pl.pallas_call(
        _prep_body,
        grid=(N // 1000,),
        in_specs=[pl.BlockSpec((2, 1000, D), lambda i: (0, i, 0))],
        out_specs=pl.BlockSpec((1000, 1), lambda i: (i, 0)),
        out_shape=jax.ShapeDtypeStruct((N, 1), jnp.float32),
    )(counts)


BR = 1000


def _mm1_body(x_ref, w_ref, h_ref):
    h_ref[...] = jnp.dot(x_ref[...], w_ref[...],
                         preferred_element_type=jnp.float32)


def _tc_mm1(x, W1):
    return pl.pallas_call(
        _mm1_body,
        grid=(N // BR,),
        in_specs=[
            pl.BlockSpec((BR, D), lambda i: (i, 0)),
            pl.BlockSpec((D, D), lambda i: (0, 0)),
        ],
        out_specs=pl.BlockSpec((BR, D), lambda i: (i, 0)),
        out_shape=jax.ShapeDtypeStruct((N, D), jnp.float32),
    )(x, W1)


def _scale_body(h1_ref, h1n_ref, dis_ref, tbl_ref):
    h = pl.program_id(0)
    sel = jnp.where(h == 0, h1_ref[...], h1n_ref[...])
    tbl_ref[...] = (dis_ref[...] * sel)[None]


def _tc_scale(h1, h1n, dis):
    return pl.pallas_call(
        _scale_body,
        grid=(2, N // BR),
        in_specs=[
            pl.BlockSpec((BR, D), lambda h, i: (i, 0)),
            pl.BlockSpec((BR, D), lambda h, i: (i, 0)),
            pl.BlockSpec((BR, 1), lambda h, i: (i, 0)),
        ],
        out_specs=pl.BlockSpec((1, BR, D), lambda h, i: (h, i, 0)),
        out_shape=jax.ShapeDtypeStruct((2, N, D), jnp.float32),
    )(h1, h1n, dis)


def _ep1mm2_body(acc_ref, tbl_ref, dis_ref, b_ref, w_ref, tbl2_ref):
    z = jnp.maximum(
        dis_ref[...] * (acc_ref[0] + tbl_ref[0]) + b_ref[...], 0.0)
    h2 = jnp.dot(z, w_ref[...], preferred_element_type=jnp.float32)
    tbl2_ref[...] = (dis_ref[...] * h2)[None]


def _tc_ep1mm2(acc, tbl, dis, b, W2):
    return pl.pallas_call(
        _ep1mm2_body,
        grid=(2, N // BR),
        in_specs=[
            pl.BlockSpec((1, BR, D), lambda h, i: (h, i, 0)),
            pl.BlockSpec((1, BR, D), lambda h, i: (h, i, 0)),
            pl.BlockSpec((BR, 1), lambda h, i: (i, 0)),
            pl.BlockSpec((1, D), lambda h, i: (0, 0)),
            pl.BlockSpec((D, D), lambda h, i: (0, 0)),
        ],
        out_specs=pl.BlockSpec((1, BR, D), lambda h, i: (h, i, 0)),
        out_shape=jax.ShapeDtypeStruct((2, N, D), jnp.float32),
    )(acc, tbl, dis, b, W2)


def _ep2_body(acc_ref, tbl_ref, dis_ref, b_ref, out_ref, s_ref):
    h = pl.program_id(0)
    i = pl.program_id(1)
    val = dis_ref[...] * (acc_ref[...] + tbl_ref[...]) + b_ref[...]
    out_ref[...] = val

    @pl.when((h == 0) & (i == 0))
    def _():
        s_ref[...] = jnp.zeros_like(s_ref)

    @pl.when(h == 0)
    def _():
        s_ref[...] += jnp.sum(val[0], axis=0, keepdims=True)

    @pl.when((h == 0) & (i == (N // BR) - 1))
    def _():
        s_ref[...] = jax.nn.sigmoid(s_ref[...] / N)


def _tc_ep2(acc, tbl, dis, b):
    return pl.pallas_call(
        _ep2_body,
        grid=(2, N // BR),
        in_specs=[
            pl.BlockSpec((1, BR, D), lambda h, i: (h, i, 0)),
            pl.BlockSpec((1, BR, D), lambda h, i: (h, i, 0)),
            pl.BlockSpec((BR, 1), lambda h, i: (i, 0)),
            pl.BlockSpec((1, D), lambda h, i: (0, 0)),
        ],
        out_specs=[
            pl.BlockSpec((1, BR, D), lambda h, i: (h, i, 0)),
            pl.BlockSpec((1, D), lambda h, i: (0, 0)),
        ],
        out_shape=[
            jax.ShapeDtypeStruct((2, N, D), jnp.float32),
            jax.ShapeDtypeStruct((1, D), jnp.float32),
        ],
    )(acc, tbl, dis, b)


def kernel(x, edge_index, W1, b1, W2, b2):
    src = edge_index[0].astype(jnp.int32)
    dst = edge_index[1].astype(jnp.int32)
    src2 = jnp.concatenate(
        [src, jnp.zeros((EPAD,), jnp.int32)]).reshape(EROWS, CH)
    pad_dst = N + jnp.arange(EPAD, dtype=jnp.int32) % (ACC_N - N)
    dst2 = jnp.concatenate([dst, pad_dst]).reshape(EROWS, CH)
    perm = jax.random.permutation(jax.random.key(42), N).astype(jnp.int32)
    perm2 = jnp.concatenate(
        [perm, jnp.zeros((NS * 8 * CH - N,), jnp.int32)]).reshape(NS * 8, CH)

    zero128 = jnp.zeros((CH, D), jnp.float32)
    ones128 = jnp.ones((CH, D), jnp.float32)
    b1r = b1.reshape(1, D)
    b2r = b2.reshape(1, D)

    counts = _sc_degree(dst2, ones128, zero128)
    dis = _tc_prep(counts)

    h1 = _tc_mm1(x, W1)
    h1n = _sc_perm(h1, perm2)
    tbl1 = _tc_scale(h1, h1n, dis)

    acc1 = _sc_msg(tbl1, src2, dst2, zero128)
    tbl2 = _tc_ep1mm2(acc1, tbl1, dis, b1r, W2)
    acc2 = _sc_msg(tbl2, src2, dst2, zero128)
    outstack, srow = _tc_ep2(acc2, tbl2, dis, b2r)

    return outstack[0], outstack[1], srow[0]


# speedup vs baseline: 2.1593x; 2.1593x over previous
import jax
import jax.numpy as jnp
from jax import lax
from jax.experimental import pallas as pl
from jax.experimental.pallas import tpu as pltpu
from jax.experimental.pallas import tpu_sc as plsc

N = 10000
E = 320000
D = 128
NC, NS = 2, 16

CH = 80
EROWS = 4096
EPAD = EROWS * CH - E
MROWS = EROWS // NS
DROWS = EROWS // (NC * NS)
JUNK = 10239

ACC_N = 10240
RPT = ACC_N // NS
HB = ACC_N // D

_mesh = plsc.VectorSubcoreMesh(core_axis_name="c", subcore_axis_name="s")


def _deg_body(dst2, ones_hbm, zero128_hbm, out, acc, idx_d, ones_v):
    c = lax.axis_index("c")
    s = lax.axis_index("s")
    w = c * NS + s

    def run(out_view):
        pltpu.sync_copy(zero128_hbm, ones_v)
        for j in range(RPT // CH):
            pltpu.sync_copy(ones_v, acc.at[pl.ds(s * RPT + j * CH, CH)])
        pltpu.sync_copy(ones_hbm, ones_v)
        pltpu.sync_copy(dst2.at[pl.ds(w * DROWS, DROWS)], idx_d)
        plsc.subcore_barrier()

        def chunk(k, carry):
            pltpu.sync_copy(ones_v, acc.at[idx_d.at[k]], add=True)
            return carry

        lax.fori_loop(0, DROWS, chunk, 0)
        plsc.subcore_barrier()
        zbuf = ones_v

        @pl.when(s < NS - 1)
        def _():
            for j in range(RPT // CH):
                r = s * RPT + j * CH
                pltpu.sync_copy(acc.at[pl.ds(r, CH)], zbuf)
                pltpu.sync_copy(zbuf, out_view.at[pl.ds(r, CH)])

        @pl.when(s == NS - 1)
        def _():
            for j in range((N - (NS - 1) * RPT) // CH):
                r = (NS - 1) * RPT + j * CH
                pltpu.sync_copy(acc.at[pl.ds(r, CH)], zbuf)
                pltpu.sync_copy(zbuf, out_view.at[pl.ds(r, CH)])

    @pl.when(c == 0)
    def _():
        run(out.at[0])

    @pl.when(c == 1)
    def _():
        run(out.at[1])


def _sc_degree(dst2, ones128, zero128):
    return pl.kernel(
        _deg_body,
        out_type=jax.ShapeDtypeStruct((NC, N, D), jnp.float32),
        mesh=_mesh,
        scratch_types=[
            pltpu.VMEM_SHARED((ACC_N, D), jnp.float32),
            pltpu.VMEM((DROWS, CH), jnp.int32),
            pltpu.VMEM((CH, D), jnp.float32),
        ],
    )(dst2, ones128, zero128)


def _perm_body(h1, perm2, out, buf, idx_v):
    c = lax.axis_index("c")
    s = lax.axis_index("s")

    def gather_chunks(nchunks):
        pltpu.sync_copy(perm2.at[pl.ds(s * 8, 8)], idx_v)
        for j in range(nchunks):
            pltpu.sync_copy(h1.at[idx_v.at[j]], buf)
            pltpu.sync_copy(buf, out.at[pl.ds(s * 8 * CH + j * CH, CH)])

    @pl.when(c == 0)
    def _():
        @pl.when(s < NS - 1)
        def _():
            gather_chunks(8)

        @pl.when(s == NS - 1)
        def _():
            gather_chunks(5)


def _sc_perm(h1, perm2):
    return pl.kernel(
        _perm_body,
        out_type=jax.ShapeDtypeStruct((N, D), jnp.float32),
        mesh=_mesh,
        scratch_types=[
            pltpu.VMEM((CH, D), jnp.float32),
            pltpu.VMEM((8, CH), jnp.int32),
        ],
    )(h1, perm2)


def _msg_body(tbl, src2, dst2, zero128_hbm, out, acc, idx_s, idx_d, buf, gsem):
    c = lax.axis_index("c")
    s = lax.axis_index("s")

    def seg(tbl_view, row0, nrows):
        pltpu.sync_copy(src2.at[pl.ds(row0, nrows)], idx_s.at[pl.ds(0, nrows)])
        pltpu.sync_copy(dst2.at[pl.ds(row0, nrows)], idx_d.at[pl.ds(0, nrows)])
        pltpu.async_copy(tbl_view.at[idx_s.at[0]], buf.at[0], gsem)

        def chunk(k, carry):
            @pl.when(k + 1 < nrows)
            def _():
                pltpu.async_copy(tbl_view.at[idx_s.at[k + 1]],
                                 buf.at[(k + 1) % 2], gsem)
            pltpu.make_async_copy(tbl_view.at[idx_s.at[k]],
                                  buf.at[k % 2], gsem).wait()
            pltpu.sync_copy(buf.at[k % 2], acc.at[idx_d.at[k]], add=True)
            return carry

        lax.fori_loop(0, nrows, chunk, 0)

    def run(tbl_view, out_view):
        zbuf = buf.at[0]
        pltpu.sync_copy(zero128_hbm, zbuf)
        for j in range(RPT // CH):
            pltpu.sync_copy(zbuf, acc.at[pl.ds(s * RPT + j * CH, CH)])
        plsc.subcore_barrier()

        for j in range(MROWS // 64):
            seg(tbl_view, s * MROWS + 64 * j, 64)

        plsc.subcore_barrier()
        @pl.when(s < NS - 1)
        def _():
            for j in range(RPT // CH):
                r = s * RPT + j * CH
                pltpu.sync_copy(acc.at[pl.ds(r, CH)], zbuf)
                pltpu.sync_copy(zbuf, out_view.at[pl.ds(r, CH)])

        @pl.when(s == NS - 1)
        def _():
            for j in range((N - (NS - 1) * RPT) // CH):
                r = (NS - 1) * RPT + j * CH
                pltpu.sync_copy(acc.at[pl.ds(r, CH)], zbuf)
                pltpu.sync_copy(zbuf, out_view.at[pl.ds(r, CH)])

    @pl.when(c == 0)
    def _():
        run(tbl.at[0], out.at[0])

    @pl.when(c == 1)
    def _():
        run(tbl.at[1], out.at[1])


def _sc_msg(tbl, src2, dst2, zero128):
    return pl.kernel(
        _msg_body,
        out_type=jax.ShapeDtypeStruct((NC, N, D), jnp.float32),
        mesh=_mesh,
        scratch_types=[
            pltpu.VMEM_SHARED((ACC_N, D), jnp.float32),
            pltpu.VMEM((64, CH), jnp.int32),
            pltpu.VMEM((64, CH), jnp.int32),
            pltpu.VMEM((2, CH, D), jnp.float32),
            pltpu.SemaphoreType.DMA,
        ],
    )(tbl, src2, dst2, zero128)


def _prep_body(cnt_ref, dis_ref):
    deg = cnt_ref[0, :, :1] + cnt_ref[1, :, :1] + 1.0
    dis_ref[...] = lax.rsqrt(deg)


def _tc_prep(counts):
    return pl.pallas_call(
        _prep_body,
        grid=(N // 1000,),
        in_specs=[pl.BlockSpec((2, 1000, D), lambda i: (0, i, 0))],
        out_specs=pl.BlockSpec((1000, 1), lambda i: (i, 0)),
        out_shape=jax.ShapeDtypeStruct((N, 1), jnp.float32),
    )(counts)


BR = 1000


def _mm1_body(x_ref, w_ref, h_ref):
    h_ref[...] = jnp.dot(x_ref[...], w_ref[...],
                         preferred_element_type=jnp.float32)


def _tc_mm1(x, W1):
    return pl.pallas_call(
        _mm1_body,
        grid=(N // BR,),
        in_specs=[
            pl.BlockSpec((BR, D), lambda i: (i, 0)),
            pl.BlockSpec((D, D), lambda i: (0, 0)),
        ],
        out_specs=pl.BlockSpec((BR, D), lambda i: (i, 0)),
        out_shape=jax.ShapeDtypeStruct((N, D), jnp.float32),
    )(x, W1)


def _scale_body(h1_ref, h1n_ref, dis_ref, tbl_ref):
    h = pl.program_id(0)
    sel = jnp.where(h == 0, h1_ref[...], h1n_ref[...])
    tbl_ref[...] = (dis_ref[...] * sel)[None]


def _tc_scale(h1, h1n, dis):
    return pl.pallas_call(
        _scale_body,
        grid=(2, N // BR),
        in_specs=[
            pl.BlockSpec((BR, D), lambda h, i: (i, 0)),
            pl.BlockSpec((BR, D), lambda h, i: (i, 0)),
            pl.BlockSpec((BR, 1), lambda h, i: (i, 0)),
        ],
        out_specs=pl.BlockSpec((1, BR, D), lambda h, i: (h, i, 0)),
        out_shape=jax.ShapeDtypeStruct((2, N, D), jnp.float32),
    )(h1, h1n, dis)


def _ep1mm2_body(acc_ref, tbl_ref, dis_ref, b_ref, w_ref, tbl2_ref):
    z = jnp.maximum(
        dis_ref[...] * (acc_ref[0] + tbl_ref[0]) + b_ref[...], 0.0)
    h2 = jnp.dot(z, w_ref[...], preferred_element_type=jnp.float32)
    tbl2_ref[...] = (dis_ref[...] * h2)[None]


def _tc_ep1mm2(acc, tbl, dis, b, W2):
    return pl.pallas_call(
        _ep1mm2_body,
        grid=(2, N // BR),
        in_specs=[
            pl.BlockSpec((1, BR, D), lambda h, i: (h, i, 0)),
            pl.BlockSpec((1, BR, D), lambda h, i: (h, i, 0)),
            pl.BlockSpec((BR, 1), lambda h, i: (i, 0)),
            pl.BlockSpec((1, D), lambda h, i: (0, 0)),
            pl.BlockSpec((D, D), lambda h, i: (0, 0)),
        ],
        out_specs=pl.BlockSpec((1, BR, D), lambda h, i: (h, i, 0)),
        out_shape=jax.ShapeDtypeStruct((2, N, D), jnp.float32),
    )(acc, tbl, dis, b, W2)


def _ep2_body(acc_ref, tbl_ref, dis_ref, b_ref, out_ref, s_ref):
    h = pl.program_id(0)
    i = pl.program_id(1)
    val = dis_ref[...] * (acc_ref[...] + tbl_ref[...]) + b_ref[...]
    out_ref[...] = val

    @pl.when((h == 0) & (i == 0))
    def _():
        s_ref[...] = jnp.zeros_like(s_ref)

    @pl.when(h == 0)
    def _():
        s_ref[...] += jnp.sum(val[0], axis=0, keepdims=True)

    @pl.when((h == 0) & (i == (N // BR) - 1))
    def _():
        s_ref[...] = jax.nn.sigmoid(s_ref[...] / N)


def _tc_ep2(acc, tbl, dis, b):
    return pl.pallas_call(
        _ep2_body,
        grid=(2, N // BR),
        in_specs=[
            pl.BlockSpec((1, BR, D), lambda h, i: (h, i, 0)),
            pl.BlockSpec((1, BR, D), lambda h, i: (h, i, 0)),
            pl.BlockSpec((BR, 1), lambda h, i: (i, 0)),
            pl.BlockSpec((1, D), lambda h, i: (0, 0)),
        ],
        out_specs=[
            pl.BlockSpec((1, BR, D), lambda h, i: (h, i, 0)),
            pl.BlockSpec((1, D), lambda h, i: (0, 0)),
        ],
        out_shape=[
            jax.ShapeDtypeStruct((2, N, D), jnp.float32),
            jax.ShapeDtypeStruct((1, D), jnp.float32),
        ],
    )(acc, tbl, dis, b)


def kernel(x, edge_index, W1, b1, W2, b2):
    src = edge_index[0].astype(jnp.int32)
    dst = edge_index[1].astype(jnp.int32)
    pad_src = jnp.arange(EPAD, dtype=jnp.int32) * 131 % N
    src2 = jnp.concatenate([src, pad_src]).reshape(EROWS, CH)
    pad_dst = N + jnp.arange(EPAD, dtype=jnp.int32) % (ACC_N - N)
    dst2 = jnp.concatenate([dst, pad_dst]).reshape(EROWS, CH)
    perm = jax.random.permutation(jax.random.key(42), N).astype(jnp.int32)
    perm2 = jnp.concatenate(
        [perm, jnp.zeros((NS * 8 * CH - N,), jnp.int32)]).reshape(NS * 8, CH)

    zero128 = jnp.zeros((CH, D), jnp.float32)
    ones128 = jnp.ones((CH, D), jnp.float32)
    b1r = b1.reshape(1, D)
    b2r = b2.reshape(1, D)

    counts = _sc_degree(dst2, ones128, zero128)
    dis = _tc_prep(counts)

    h1 = _tc_mm1(x, W1)
    h1n = _sc_perm(h1, perm2)
    tbl1 = _tc_scale(h1, h1n, dis)

    acc1 = _sc_msg(tbl1, src2, dst2, zero128)
    tbl2 = _tc_ep1mm2(acc1, tbl1, dis, b1r, W2)
    acc2 = _sc_msg(tbl2, src2, dst2, zero128)
    outstack, srow = _tc_ep2(acc2, tbl2, dis, b2r)

    return outstack[0], outstack[1], srow[0]
